# baseline (speedup 1.0000x reference)
import functools

import jax
import jax.numpy as jnp
from jax import lax
from jax.experimental import pallas as pl
from jax.experimental.pallas import tpu as pltpu
from jax.experimental.pallas import tpu_sc as plsc

B, L, D = 16384, 50, 16
N = B * L
NC, NS = 2, 16
NW = NC * NS
N_W = N // NW
C = 128
NCH = N_W // C
G = 10
GC = G * C
NG = NCH // G


@jax.jit
def _sc_gather(idx, table):
    mesh = plsc.VectorSubcoreMesh(core_axis_name="c", subcore_axis_name="s")

    @functools.partial(
        pl.kernel,
        mesh=mesh,
        out_type=jax.ShapeDtypeStruct((NW, N_W, D), jnp.float32),
        scratch_types=[
            pltpu.VMEM((NCH, C), jnp.int32),
            pltpu.VMEM((2, GC, D), jnp.float32),
            pltpu.SemaphoreType.DMA,
            pltpu.SemaphoreType.DMA,
            pltpu.SemaphoreType.DMA,
            pltpu.SemaphoreType.DMA,
        ],
        compiler_params=pltpu.CompilerParams(
            use_tc_tiling_on_sc=False, needs_layout_passes=False
        ),
    )
    def k(idx_hbm, table_hbm, out_hbm, idx_v, rows_v, g0, g1, s0, s1):
        wid = lax.axis_index("s") * NC + lax.axis_index("c")
        gsem = (g0, g1)
        ssem = (s0, s1)
        pltpu.sync_copy(idx_hbm.at[wid], idx_v)

        def fire(g, b):
            for j in range(G):
                pltpu.async_copy(
                    table_hbm.at[idx_v.at[g * G + j]],
                    rows_v.at[b, pl.ds(j * C, C), :],
                    gsem[b],
                )

        def drain_gather(b):
            pltpu.make_async_copy(
                table_hbm.at[pl.ds(0, GC), :], rows_v.at[b], gsem[b]
            ).wait()

        def store(g, b):
            pltpu.async_copy(
                rows_v.at[b], out_hbm.at[wid, pl.ds(g * GC, GC), :], ssem[b]
            )

        def wait_store(b):
            pltpu.make_async_copy(
                rows_v.at[b], out_hbm.at[wid, pl.ds(0, GC), :], ssem[b]
            ).wait()

        fire(0, 0)
        drain_gather(0)
        fire(1, 1)
        store(0, 0)

        @pl.loop(1, NG - 2, step=2)
        def _pair(g):
            drain_gather(1)
            wait_store(0)
            fire(g + 1, 0)
            store(g, 1)
            drain_gather(0)
            wait_store(1)
            fire(g + 2, 1)
            store(g + 1, 0)

        drain_gather(1)
        store(NG - 1, 1)
        wait_store(0)
        wait_store(1)

    return k(idx, table)


def kernel(indices, table):
    idx = jnp.asarray(indices, jnp.int32).reshape(NW, NCH, C)
    out = _sc_gather(idx, table)
    return out.reshape(B, L, D)


# speedup vs baseline: 5.2020x; 5.2020x over previous
import functools

import jax
import jax.numpy as jnp
from jax import lax
from jax.experimental import pallas as pl
from jax.experimental.pallas import tpu as pltpu
from jax.experimental.pallas import tpu_sc as plsc

B, L, D = 16384, 50, 16
V = 1000000
NC, NS = 2, 16
DG = D // NC
BT = B // NS
C = 128
NCH = BT // C
RING = 10


@jax.jit
def _sc_gather(idx_t, table_t):
    mesh = plsc.VectorSubcoreMesh(core_axis_name="c", subcore_axis_name="s")

    @functools.partial(
        pl.kernel,
        mesh=mesh,
        out_type=jax.ShapeDtypeStruct((L, D, B), jnp.float32),
        scratch_types=[
            pltpu.VMEM((L * BT,), jnp.int32),
            pltpu.VMEM((RING * BT,), jnp.float32),
            pltpu.VMEM_SHARED((V,), jnp.float32),
            pltpu.SemaphoreType.DMA,
            [pltpu.SemaphoreType.DMA] * RING,
            [pltpu.SemaphoreType.DMA] * RING,
        ],
        compiler_params=pltpu.CompilerParams(use_tc_tiling_on_sc=True),
    )
    def k(idx_hbm, table_hbm, out_hbm, idx_v, res_v, col_v, ssem, gsems, stsems):
        c = lax.axis_index("c")
        s = lax.axis_index("s")

        @pl.loop(0, L)
        def _ld(l):
            pltpu.async_copy(
                idx_hbm.at[l, pl.ds(s * BT, BT)],
                idx_v.at[pl.ds(l * BT, BT)],
                ssem,
            )

        @pl.loop(0, L)
        def _ld_wait(l):
            pltpu.make_async_copy(
                idx_hbm.at[0, pl.ds(0, BT)], idx_v.at[pl.ds(0, BT)], ssem
            ).wait()

        for j in range(DG):
            @pl.when(s == NS - 1)
            def _stage():
                pltpu.async_copy(table_hbm.at[c * DG + j], col_v, ssem)
                pltpu.make_async_copy(table_hbm.at[0], col_v, ssem).wait()

            plsc.subcore_barrier()

            @pl.loop(0, L, step=RING)
            def _rows(g0):
                for r in range(RING):
                    @pl.when(g0 > 0)
                    def _w():
                        pltpu.make_async_copy(
                            res_v.at[pl.ds(0, BT)],
                            out_hbm.at[0, 0, pl.ds(0, BT)],
                            stsems[r],
                        ).wait()

                    for j8 in range(NCH):
                        pltpu.async_copy(
                            col_v.at[idx_v.at[pl.ds((g0 + r) * BT + j8 * C, C)]],
                            res_v.at[pl.ds(r * BT + j8 * C, C)],
                            gsems[r],
                        )

                for r in range(RING):
                    pltpu.make_async_copy(
                        idx_hbm.at[0, pl.ds(0, BT)],
                        res_v.at[pl.ds(0, BT)],
                        gsems[r],
                    ).wait()
                    pltpu.async_copy(
                        res_v.at[pl.ds(r * BT, BT)],
                        out_hbm.at[g0 + r, c * DG + j, pl.ds(s * BT, BT)],
                        stsems[r],
                    )

            for r in range(RING):
                pltpu.make_async_copy(
                    res_v.at[pl.ds(0, BT)],
                    out_hbm.at[0, 0, pl.ds(0, BT)],
                    stsems[r],
                ).wait()

            plsc.subcore_barrier()

    return k(idx_t, table_t)


def kernel(indices, table):
    idx_t = jnp.swapaxes(jnp.asarray(indices, jnp.int32), 0, 1)
    table_t = jnp.swapaxes(table, 0, 1)
    out_t = _sc_gather(idx_t, table_t)
    return jnp.transpose(out_t, (2, 0, 1))


# speedup vs baseline: 5.4167x; 1.0413x over previous
import functools

import jax
import jax.numpy as jnp
from jax import lax
from jax.experimental import pallas as pl
from jax.experimental.pallas import tpu as pltpu
from jax.experimental.pallas import tpu_sc as plsc

B, L, D = 16384, 50, 16
V = 1000000
NC, NS = 2, 16
DG = D // NC
BT = B // NS
C = 1024
NCH = BT // C
RING = 10


@jax.jit
def _sc_gather(idx_t, table_t):
    mesh = plsc.VectorSubcoreMesh(core_axis_name="c", subcore_axis_name="s")

    @functools.partial(
        pl.kernel,
        mesh=mesh,
        out_type=jax.ShapeDtypeStruct((L, D, B), jnp.float32),
        scratch_types=[
            pltpu.VMEM((L * BT,), jnp.int32),
            pltpu.VMEM((RING * BT,), jnp.float32),
            pltpu.VMEM_SHARED((V,), jnp.float32),
            pltpu.SemaphoreType.DMA,
            [pltpu.SemaphoreType.DMA] * RING,
            [pltpu.SemaphoreType.DMA] * RING,
        ],
        compiler_params=pltpu.CompilerParams(use_tc_tiling_on_sc=True),
    )
    def k(idx_hbm, table_hbm, out_hbm, idx_v, res_v, col_v, ssem, gsems, stsems):
        c = lax.axis_index("c")
        s = lax.axis_index("s")

        @pl.loop(0, L)
        def _ld(l):
            pltpu.async_copy(
                idx_hbm.at[l, pl.ds(s * BT, BT)],
                idx_v.at[pl.ds(l * BT, BT)],
                ssem,
            )

        @pl.loop(0, L)
        def _ld_wait(l):
            pltpu.make_async_copy(
                idx_hbm.at[0, pl.ds(0, BT)], idx_v.at[pl.ds(0, BT)], ssem
            ).wait()

        for j in range(DG):
            @pl.when(s == NS - 1)
            def _stage():
                pltpu.async_copy(table_hbm.at[c * DG + j], col_v, ssem)
                pltpu.make_async_copy(table_hbm.at[0], col_v, ssem).wait()

            plsc.subcore_barrier()

            @pl.loop(0, L, step=RING)
            def _rows(g0):
                for r in range(RING):
                    @pl.when(g0 > 0)
                    def _w():
                        pltpu.make_async_copy(
                            res_v.at[pl.ds(0, BT)],
                            out_hbm.at[0, 0, pl.ds(0, BT)],
                            stsems[r],
                        ).wait()

                    for j8 in range(NCH):
                        pltpu.async_copy(
                            col_v.at[idx_v.at[pl.ds((g0 + r) * BT + j8 * C, C)]],
                            res_v.at[pl.ds(r * BT + j8 * C, C)],
                            gsems[r],
                        )

                for r in range(RING):
                    pltpu.make_async_copy(
                        idx_hbm.at[0, pl.ds(0, BT)],
                        res_v.at[pl.ds(0, BT)],
                        gsems[r],
                    ).wait()
                    pltpu.async_copy(
                        res_v.at[pl.ds(r * BT, BT)],
                        out_hbm.at[g0 + r, c * DG + j, pl.ds(s * BT, BT)],
                        stsems[r],
                    )

            for r in range(RING):
                pltpu.make_async_copy(
                    res_v.at[pl.ds(0, BT)],
                    out_hbm.at[0, 0, pl.ds(0, BT)],
                    stsems[r],
                ).wait()

            plsc.subcore_barrier()

    return k(idx_t, table_t)


def kernel(indices, table):
    idx_t = jnp.swapaxes(jnp.asarray(indices, jnp.int32), 0, 1)
    table_t = jnp.swapaxes(table, 0, 1)
    out_t = _sc_gather(idx_t, table_t)
    return jnp.transpose(out_t, (2, 0, 1))


# speedup vs baseline: 5.4872x; 1.0130x over previous
import functools

import jax
import jax.numpy as jnp
from jax import lax
from jax.experimental import pallas as pl
from jax.experimental.pallas import tpu as pltpu
from jax.experimental.pallas import tpu_sc as plsc

B, L, D = 16384, 50, 16
V = 1000000
NC, NS = 2, 16
DG = D // NC
BT = B // NS
RING = 10


@jax.jit
def _sc_gather(idx_t, table_t):
    mesh = plsc.VectorSubcoreMesh(core_axis_name="c", subcore_axis_name="s")

    @functools.partial(
        pl.kernel,
        mesh=mesh,
        out_type=jax.ShapeDtypeStruct((L, D, B), jnp.float32),
        scratch_types=[
            pltpu.VMEM((L * BT,), jnp.int32),
            pltpu.VMEM((RING * BT,), jnp.float32),
            pltpu.VMEM_SHARED((V,), jnp.float32),
            pltpu.SemaphoreType.DMA,
            [pltpu.SemaphoreType.DMA] * RING,
            [pltpu.SemaphoreType.DMA] * RING,
        ],
        compiler_params=pltpu.CompilerParams(use_tc_tiling_on_sc=True),
    )
    def k(idx_hbm, table_hbm, out_hbm, idx_v, res_v, col_v, ssem, gsems, stsems):
        c = lax.axis_index("c")
        s = lax.axis_index("s")

        @pl.loop(0, L)
        def _ld(l):
            pltpu.async_copy(
                idx_hbm.at[l, pl.ds(s * BT, BT)],
                idx_v.at[pl.ds(l * BT, BT)],
                ssem,
            )

        @pl.when(s == NS - 1)
        def _stage0():
            pltpu.async_copy(table_hbm.at[c * DG], col_v, ssem)

        @pl.loop(0, L)
        def _ld_wait(l):
            pltpu.make_async_copy(
                idx_hbm.at[0, pl.ds(0, BT)], idx_v.at[pl.ds(0, BT)], ssem
            ).wait()

        for j in range(DG):
            @pl.when(s == NS - 1)
            def _stage_wait():
                pltpu.make_async_copy(table_hbm.at[0], col_v, ssem).wait()

            plsc.subcore_barrier()

            @pl.loop(0, L, step=RING)
            def _rows(g0):
                for r in range(RING):
                    if j == 0:
                        @pl.when(g0 > 0)
                        def _w():
                            pltpu.make_async_copy(
                                res_v.at[pl.ds(0, BT)],
                                out_hbm.at[0, 0, pl.ds(0, BT)],
                                stsems[r],
                            ).wait()
                    else:
                        pltpu.make_async_copy(
                            res_v.at[pl.ds(0, BT)],
                            out_hbm.at[0, 0, pl.ds(0, BT)],
                            stsems[r],
                        ).wait()

                    pltpu.async_copy(
                        col_v.at[idx_v.at[pl.ds((g0 + r) * BT, BT)]],
                        res_v.at[pl.ds(r * BT, BT)],
                        gsems[r],
                    )

                for r in range(RING):
                    pltpu.make_async_copy(
                        idx_hbm.at[0, pl.ds(0, BT)],
                        res_v.at[pl.ds(0, BT)],
                        gsems[r],
                    ).wait()
                    pltpu.async_copy(
                        res_v.at[pl.ds(r * BT, BT)],
                        out_hbm.at[g0 + r, c * DG + j, pl.ds(s * BT, BT)],
                        stsems[r],
                    )

            plsc.subcore_barrier()

            if j + 1 < DG:
                @pl.when(s == NS - 1)
                def _stage_next():
                    pltpu.async_copy(table_hbm.at[c * DG + j + 1], col_v, ssem)

        for r in range(RING):
            pltpu.make_async_copy(
                res_v.at[pl.ds(0, BT)],
                out_hbm.at[0, 0, pl.ds(0, BT)],
                stsems[r],
            ).wait()

    return k(idx_t, table_t)


def kernel(indices, table):
    idx_t = jnp.swapaxes(jnp.asarray(indices, jnp.int32), 0, 1)
    table_t = jnp.swapaxes(table, 0, 1)
    out_t = _sc_gather(idx_t, table_t)
    return jnp.transpose(out_t, (2, 0, 1))
